# baseline (speedup 1.0000x reference)
import jax
import jax.numpy as jnp
from jax import lax
from jax.experimental import pallas as pl
from jax.experimental.pallas import tpu as pltpu

_N = 8192
_D = 4096
_E = 64
_R = 16
_C = _E * _R
_BT = 512


_CH = 256


def _body(x_ref, sf_ref, w_ref, sel_ref, sh_ref):
    @pl.when(pl.program_id(0) == 0)
    def _():
        sh_ref[...] = sf_ref[...].astype(jnp.bfloat16)

    sh = sh_ref[...]
    for c in range(_BT // _CH):
        sl = pl.ds(c * _CH, _CH)
        x = x_ref[sl, :]
        nrm = jnp.sqrt(jnp.sum(x * x, axis=1, keepdims=True))
        xn = x * (1.0 / jnp.maximum(nrm, 1e-12))

        xh = xn.astype(jnp.bfloat16)
        proj = jnp.dot(xh, sh, preferred_element_type=jnp.float32)

        p2 = proj * proj
        acc = p2[:, 0:128]
        for k in range(1, 8):
            acc = acc + p2[:, k * 128:(k + 1) * 128]
        o2 = acc[:, 0:64] + acc[:, 64:128]

        logits = jnp.sqrt(o2) * -10.0
        m = jnp.max(logits, axis=1, keepdims=True)
        e = jnp.exp(logits - m)
        w = e / jnp.sum(e, axis=1, keepdims=True)
        w_ref[sl, :] = w

        iota = lax.broadcasted_iota(jnp.int32, (_CH, _E), 1)
        m1 = jnp.max(w, axis=1, keepdims=True)
        i1 = jnp.min(jnp.where(w == m1, iota, _E), axis=1, keepdims=True)
        w2 = jnp.where(iota == i1, -1.0, w)
        m2 = jnp.max(w2, axis=1, keepdims=True)
        i2 = jnp.min(jnp.where(w2 == m2, iota, _E), axis=1, keepdims=True)
        sel_ref[sl, :] = jnp.concatenate([i1, i2], axis=1)


def _route(x, sh):
    n = x.shape[0]
    grid = (n // _BT,)
    return pl.pallas_call(
        _body,
        grid=grid,
        in_specs=[
            pl.BlockSpec((_BT, _D), lambda i: (i, 0)),
            pl.BlockSpec((_D, _C), lambda i: (0, 0)),
        ],
        out_specs=[
            pl.BlockSpec((_BT, _E), lambda i: (i, 0)),
            pl.BlockSpec((_BT, 2), lambda i: (i, 0)),
        ],
        out_shape=[
            jax.ShapeDtypeStruct((n, _E), jnp.float32),
            jax.ShapeDtypeStruct((n, 2), jnp.int32),
        ],
        scratch_shapes=[pltpu.VMEM((_D, _C), jnp.bfloat16)],
        compiler_params=pltpu.CompilerParams(
            dimension_semantics=("arbitrary",),
        ),
    )(x, sh)


def kernel(x, expert_subspaces):
    s = expert_subspaces.transpose(1, 2, 0).reshape(_D, _C)

    return _route(x, s)


# speedup vs baseline: 1.2314x; 1.2314x over previous
import jax
import jax.numpy as jnp
from jax import lax
from jax.experimental import pallas as pl
from jax.experimental.pallas import tpu as pltpu

_N = 8192
_D = 4096
_E = 64
_R = 16
_C = _E * _R
_BT = 512
_CH = 256


def _body(x_ref, sh_ref, wt_ref, selt_ref):
    sh = sh_ref[...]
    for c in range(_BT // _CH):
        sl = pl.ds(c * _CH, _CH)
        x = x_ref[sl, :]
        nrm = jnp.sqrt(jnp.sum(x * x, axis=1, keepdims=True))
        xn = x * (1.0 / jnp.maximum(nrm, 1e-12))

        xh = xn.astype(jnp.bfloat16)
        proj = jnp.dot(xh, sh, preferred_element_type=jnp.float32)

        p2 = proj * proj
        acc = p2[:, 0:128]
        for k in range(1, 8):
            acc = acc + p2[:, k * 128:(k + 1) * 128]
        o2 = acc[:, 0:64] + acc[:, 64:128]

        logits = jnp.sqrt(o2) * -10.0
        m = jnp.max(logits, axis=1, keepdims=True)
        e = jnp.exp(logits - m)
        w = e / jnp.sum(e, axis=1, keepdims=True)
        wt_ref[:, sl] = w.T

        iota = lax.broadcasted_iota(jnp.int32, (_CH, _E), 1)
        m1 = jnp.max(w, axis=1, keepdims=True)
        i1 = jnp.min(jnp.where(w == m1, iota, _E), axis=1, keepdims=True)
        w2 = jnp.where(iota == i1, -1.0, w)
        m2 = jnp.max(w2, axis=1, keepdims=True)
        i2 = jnp.min(jnp.where(w2 == m2, iota, _E), axis=1, keepdims=True)
        selt_ref[:, sl] = jnp.concatenate([i1, i2], axis=1).T


def _route(x, sh):
    n = x.shape[0]
    grid = (n // _BT,)
    wt, selt = pl.pallas_call(
        _body,
        grid=grid,
        in_specs=[
            pl.BlockSpec((_BT, _D), lambda i: (i, 0)),
            pl.BlockSpec((_D, _C), lambda i: (0, 0)),
        ],
        out_specs=[
            pl.BlockSpec((_E, _BT), lambda i: (0, i)),
            pl.BlockSpec((2, _BT), lambda i: (0, i)),
        ],
        out_shape=[
            jax.ShapeDtypeStruct((_E, n), jnp.float32),
            jax.ShapeDtypeStruct((2, n), jnp.int32),
        ],
        compiler_params=pltpu.CompilerParams(
            dimension_semantics=("parallel",),
        ),
    )(x, sh)
    return wt.T, selt.T


def kernel(x, expert_subspaces):
    s = expert_subspaces.transpose(1, 2, 0).reshape(_D, _C)
    sh = s.astype(jnp.bfloat16)

    return _route(x, sh)


# speedup vs baseline: 1.3962x; 1.1338x over previous
import jax
import jax.numpy as jnp
from jax import lax
from jax.experimental import pallas as pl
from jax.experimental.pallas import tpu as pltpu

_N = 8192
_D = 4096
_E = 64
_R = 16
_C = _E * _R
_BT = 512
_CH = 256


def _body(x_ref, st_ref, wt_ref, selt_ref):
    st = st_ref[...]
    for c in range(_BT // _CH):
        sl = pl.ds(c * _CH, _CH)
        x = x_ref[sl, :]
        nrm = jnp.sqrt(jnp.sum(x * x, axis=1, keepdims=True))
        xn = x * (1.0 / jnp.maximum(nrm, 1e-12))

        xh = xn.astype(jnp.bfloat16)
        projt = lax.dot_general(
            st, xh, (((1,), (1,)), ((), ())),
            preferred_element_type=jnp.float32,
        )

        p2 = projt * projt
        o2t = jnp.sum(p2.reshape(_E, _R, _CH), axis=1)

        logits = jnp.sqrt(o2t) * -10.0
        m = jnp.max(logits, axis=0, keepdims=True)
        e = jnp.exp(logits - m)
        w = e / jnp.sum(e, axis=0, keepdims=True)
        wt_ref[:, sl] = w

        iota = lax.broadcasted_iota(jnp.int32, (_E, _CH), 0)
        m1 = jnp.max(w, axis=0, keepdims=True)
        i1 = jnp.min(jnp.where(w == m1, iota, _E), axis=0, keepdims=True)
        w2 = jnp.where(iota == i1, -1.0, w)
        m2 = jnp.max(w2, axis=0, keepdims=True)
        i2 = jnp.min(jnp.where(w2 == m2, iota, _E), axis=0, keepdims=True)
        selt_ref[:, sl] = jnp.concatenate([i1, i2], axis=0)


def _route(x, sh):
    n = x.shape[0]
    grid = (n // _BT,)
    wt, selt = pl.pallas_call(
        _body,
        grid=grid,
        in_specs=[
            pl.BlockSpec((_BT, _D), lambda i: (i, 0)),
            pl.BlockSpec((_C, _D), lambda i: (0, 0)),
        ],
        out_specs=[
            pl.BlockSpec((_E, _BT), lambda i: (0, i)),
            pl.BlockSpec((2, _BT), lambda i: (0, i)),
        ],
        out_shape=[
            jax.ShapeDtypeStruct((_E, n), jnp.float32),
            jax.ShapeDtypeStruct((2, n), jnp.int32),
        ],
        compiler_params=pltpu.CompilerParams(
            dimension_semantics=("parallel",),
        ),
    )(x, sh)
    return wt.T, selt.T


def kernel(x, expert_subspaces):
    st = expert_subspaces.transpose(0, 2, 1).reshape(_C, _D)
    sh = st.astype(jnp.bfloat16)

    return _route(x, sh)


# speedup vs baseline: 1.4729x; 1.0550x over previous
import jax
import jax.numpy as jnp
from jax import lax
from jax.experimental import pallas as pl
from jax.experimental.pallas import tpu as pltpu

_N = 8192
_D = 4096
_E = 64
_R = 16
_C = _E * _R
_BT = 512
_CH = 256


def _body(x_ref, sf_ref, wt_ref, selt_ref, st_ref):
    @pl.when(pl.program_id(0) == 0)
    def _():
        st_ref[...] = sf_ref[...].astype(jnp.bfloat16)

    st = st_ref[...]
    for c in range(_BT // _CH):
        sl = pl.ds(c * _CH, _CH)
        x = x_ref[sl, :]
        nrm = jnp.sqrt(jnp.sum(x * x, axis=1, keepdims=True))
        xn = x * (1.0 / jnp.maximum(nrm, 1e-12))

        xh = xn.astype(jnp.bfloat16)
        projt = lax.dot_general(
            st, xh, (((1,), (1,)), ((), ())),
            preferred_element_type=jnp.float32,
        )

        p2 = projt * projt
        o2t = jnp.sum(p2.reshape(_E, _R, _CH), axis=1)

        logits = jnp.sqrt(o2t) * -10.0
        m = jnp.max(logits, axis=0, keepdims=True)
        e = jnp.exp(logits - m)
        w = e / jnp.sum(e, axis=0, keepdims=True)
        wt_ref[:, sl] = w

        iota = lax.broadcasted_iota(jnp.int32, (_E, _CH), 0)
        m1 = jnp.max(w, axis=0, keepdims=True)
        i1 = jnp.min(jnp.where(w == m1, iota, _E), axis=0, keepdims=True)
        w2 = jnp.where(iota == i1, -1.0, w)
        m2 = jnp.max(w2, axis=0, keepdims=True)
        i2 = jnp.min(jnp.where(w2 == m2, iota, _E), axis=0, keepdims=True)
        selt_ref[:, sl] = jnp.concatenate([i1, i2], axis=0)


def _route(x, sh):
    n = x.shape[0]
    grid = (n // _BT,)
    wt, selt = pl.pallas_call(
        _body,
        grid=grid,
        in_specs=[
            pl.BlockSpec((_BT, _D), lambda i: (i, 0)),
            pl.BlockSpec((_C, _D), lambda i: (0, 0)),
        ],
        out_specs=[
            pl.BlockSpec((_E, _BT), lambda i: (0, i)),
            pl.BlockSpec((2, _BT), lambda i: (0, i)),
        ],
        out_shape=[
            jax.ShapeDtypeStruct((_E, n), jnp.float32),
            jax.ShapeDtypeStruct((2, n), jnp.int32),
        ],
        scratch_shapes=[pltpu.VMEM((_C, _D), jnp.bfloat16)],
        compiler_params=pltpu.CompilerParams(
            dimension_semantics=("arbitrary",),
        ),
    )(x, sh)
    return wt.T, selt.T


def kernel(x, expert_subspaces):
    st = expert_subspaces.transpose(0, 2, 1).reshape(_C, _D)

    return _route(x, st)
